# baseline (speedup 1.0000x reference)
import functools

import jax
import jax.numpy as jnp
from jax import lax
from jax.experimental import pallas as pl
from jax.experimental.pallas import tpu as pltpu
from jax.experimental.pallas import tpu_sc as plsc

N = 10000
E = 320000
D = 128
G = 64

NC = 2
NS = 16
NW = NC * NS
EPW = E // NW
K = 40
NCHUNK = EPW // K
B = 5
NPAD = 10240
RPT = NPAD // NS


def _sc_agg(x, src_flat, dst_flat):
    mesh = plsc.VectorSubcoreMesh(core_axis_name="c", subcore_axis_name="s")

    @functools.partial(
        pl.kernel,
        mesh=mesh,
        out_type=jax.ShapeDtypeStruct((NC, NPAD, D), jnp.float32),
        scratch_types=(
            [pltpu.VMEM((B * K,), jnp.int32)]
            + [pltpu.VMEM((B, K), jnp.int32)]
            + [pltpu.VMEM((K, D), jnp.float32) for _ in range(B)]
            + [pltpu.VMEM_SHARED((NPAD, D), jnp.float32)]
            + [pltpu.SemaphoreType.DMA for _ in range(4 * B)]
        ),
    )
    def agg_kernel(x_hbm, src_hbm, dst_hbm, out_hbm, sidx_v, didx_v, *rest):
        rows = rest[0:B]
        acc_sh = rest[B]
        gsem = rest[B + 1:B + 1 + B]
        ssem = rest[B + 1 + B:B + 1 + 2 * B]
        dsem = rest[B + 1 + 2 * B:B + 1 + 3 * B]
        isem = rest[B + 1 + 3 * B:B + 1 + 4 * B]
        c = lax.axis_index("c")
        s = lax.axis_index("s")
        wid = c * NS + s

        z16 = jnp.zeros((16,), jnp.float32)

        def zb_body(i, carry):
            r = i // 8
            col = (i % 8) * 16
            rows[0][r, pl.ds(col, 16)] = z16
            rows[1][r, pl.ds(col, 16)] = z16
            return carry

        lax.fori_loop(0, K * 8, zb_body, 0, unroll=8)

        def zcopy_body(j, carry):
            pltpu.sync_copy(rows[0], acc_sh.at[pl.ds(s * RPT + (2 * j) * K, K)])
            pltpu.sync_copy(rows[1], acc_sh.at[pl.ds(s * RPT + (2 * j + 1) * K, K)])
            return carry

        lax.fori_loop(0, RPT // (2 * K), zcopy_body, 0)
        plsc.subcore_barrier()

        def i_start(i, b):
            pltpu.async_copy(src_hbm.at[pl.ds(wid * EPW + i * K, K)],
                             sidx_v.at[pl.ds(b * K, K)], isem[b])

        def i_wait(i, b):
            pltpu.make_async_copy(src_hbm.at[pl.ds(wid * EPW + i * K, K)],
                                  sidx_v.at[pl.ds(b * K, K)], isem[b]).wait()

        def g_start(i, b):
            pltpu.async_copy(x_hbm.at[sidx_v.at[pl.ds(b * K, K)]],
                             rows[b], gsem[b])
            pltpu.async_copy(dst_hbm.at[pl.ds(wid * EPW + i * K, K)],
                             didx_v.at[b], dsem[b])

        def g_wait(i, b):
            pltpu.make_async_copy(x_hbm.at[sidx_v.at[pl.ds(b * K, K)]],
                                  rows[b], gsem[b]).wait()
            pltpu.make_async_copy(dst_hbm.at[pl.ds(wid * EPW + i * K, K)],
                                  didx_v.at[b], dsem[b]).wait()

        def s_start(i, b):
            pltpu.async_copy(rows[b], acc_sh.at[didx_v.at[b]], ssem[b],
                             add=True)

        def s_wait(i, b):
            pltpu.make_async_copy(rows[b], acc_sh.at[didx_v.at[b]],
                                  ssem[b]).wait()

        for t in range(B - 2):
            i_start(t, t)
            i_wait(t, t)
            g_start(t, t)
        i_start(B - 2, B - 2)

        def ring_body(j, carry):
            for t in range(B):
                i = B * j + t
                b = t
                f = (t + B - 1) % B
                g = (t + B - 2) % B
                g_wait(i, b)
                s_start(i, b)

                @pl.when(jnp.logical_and(i > 0, i < NCHUNK - B + 1))
                def _():
                    s_wait(i - 1, f)

                @pl.when(i < NCHUNK - B + 1)
                def _():
                    i_start(i + B - 1, f)

                @pl.when(i < NCHUNK - B + 2)
                def _():
                    i_wait(i + B - 2, g)
                    g_start(i + B - 2, g)
            return carry

        lax.fori_loop(0, NCHUNK // B, ring_body, 0)
        for t in range(B):
            i = NCHUNK - B + t
            s_wait(i, i % B)
        plsc.subcore_barrier()

        pltpu.sync_copy(acc_sh.at[pl.ds(s * RPT, RPT)],
                        out_hbm.at[c, pl.ds(s * RPT, RPT)])

    return agg_kernel(x, src_flat, dst_flat)[:, :N, :]


def _tc_layer_body(eps_ref, x_ref, p_ref, wa_ref, ba_ref, wb_ref, bb_ref,
                   g_ref, beta_ref, o_ref):
    h = eps_ref[0, 0] * x_ref[...] + p_ref[0] + p_ref[1]
    t = jnp.maximum(
        jnp.dot(h, wa_ref[...], preferred_element_type=jnp.float32)
        + ba_ref[...], 0.0)
    u = (jnp.dot(t, wb_ref[...], preferred_element_type=jnp.float32)
         + bb_ref[...])
    m = jnp.mean(u, axis=0, keepdims=True)
    v = jnp.mean((u - m) ** 2, axis=0, keepdims=True)
    o_ref[...] = jnp.maximum(
        (u - m) * lax.rsqrt(v + 1e-5) * g_ref[...] + beta_ref[...], 0.0)


def _tc_layer(x, p, eps, Wa, ba, Wb, bb, g, beta):
    eps_s = jnp.reshape(1.0 + eps, (1, 1))
    return pl.pallas_call(
        _tc_layer_body,
        out_shape=jax.ShapeDtypeStruct((N, D), jnp.float32),
    )(eps_s, x, p, Wa, ba.reshape(1, D), Wb, bb.reshape(1, D),
      g.reshape(1, D), beta.reshape(1, D))


def _tc_final_body(eps_ref, x_ref, p_ref, wa_ref, ba_ref, wb_ref, bb_ref,
                   batch_ref, o_ref):
    h = eps_ref[0, 0] * x_ref[...] + p_ref[0] + p_ref[1]
    t = jnp.maximum(
        jnp.dot(h, wa_ref[...], preferred_element_type=jnp.float32)
        + ba_ref[...], 0.0)
    u = (jnp.dot(t, wb_ref[...], preferred_element_type=jnp.float32)
         + bb_ref[...])
    gids = lax.broadcasted_iota(jnp.int32, (N, G), 1)
    onehot = (batch_ref[...] == gids).astype(jnp.float32)
    o_ref[...] = lax.dot_general(
        onehot, u, (((0,), (0,)), ((), ())),
        preferred_element_type=jnp.float32)


def _tc_final(x, p, eps, Wa, ba, Wb, bb, batch):
    eps_s = jnp.reshape(1.0 + eps, (1, 1))
    return pl.pallas_call(
        _tc_final_body,
        out_shape=jax.ShapeDtypeStruct((G, D), jnp.float32),
    )(eps_s, x, p, Wa, ba.reshape(1, D), Wb, bb.reshape(1, D),
      batch.reshape(N, 1))


def kernel(x, edge_index, batch,
           eps0, W0a, b0a, W0b, b0b, g0, beta0,
           eps1, W1a, b1a, W1b, b1b, g1, beta1,
           eps2, W2a, b2a, W2b, b2b):
    src = edge_index[0]
    dst = edge_index[1]

    p0 = _sc_agg(x, src, dst)
    x1 = _tc_layer(x, p0, eps0, W0a, b0a, W0b, b0b, g0, beta0)
    p1 = _sc_agg(x1, src, dst)
    x2 = _tc_layer(x1, p1, eps1, W1a, b1a, W1b, b1b, g1, beta1)
    p2 = _sc_agg(x2, src, dst)
    return _tc_final(x2, p2, eps2, W2a, b2a, W2b, b2b, batch)


# speedup vs baseline: 1.0534x; 1.0534x over previous
import functools

import jax
import jax.numpy as jnp
from jax import lax
from jax.experimental import pallas as pl
from jax.experimental.pallas import tpu as pltpu
from jax.experimental.pallas import tpu_sc as plsc

N = 10000
E = 320000
D = 128
G = 64

NC = 2
NS = 16
NW = NC * NS
EPW = E // NW
K = 40
NCHUNK = EPW // K
B = 5
NPAD = 10240
RPT = NPAD // NS


def _sc_agg(x, src_flat, dst_flat):
    mesh = plsc.VectorSubcoreMesh(core_axis_name="c", subcore_axis_name="s")

    @functools.partial(
        pl.kernel,
        mesh=mesh,
        out_type=jax.ShapeDtypeStruct((NC, NPAD, D), jnp.float32),
        scratch_types=(
            [pltpu.VMEM((B * K,), jnp.int32)]
            + [pltpu.VMEM((B, K), jnp.int32)]
            + [pltpu.VMEM((K, D), jnp.float32) for _ in range(B)]
            + [pltpu.VMEM_SHARED((NPAD, D), jnp.float32)]
            + [pltpu.SemaphoreType.DMA for _ in range(4 * B)]
        ),
    )
    def agg_kernel(x_hbm, src_hbm, dst_hbm, out_hbm, sidx_v, didx_v, *rest):
        rows = rest[0:B]
        acc_sh = rest[B]
        gsem = rest[B + 1:B + 1 + B]
        ssem = rest[B + 1 + B:B + 1 + 2 * B]
        dsem = rest[B + 1 + 2 * B:B + 1 + 3 * B]
        isem = rest[B + 1 + 3 * B:B + 1 + 4 * B]
        c = lax.axis_index("c")
        s = lax.axis_index("s")
        wid = c * NS + s

        z16 = jnp.zeros((16,), jnp.float32)

        def zb_body(i, carry):
            r = i // 8
            col = (i % 8) * 16
            rows[0][r, pl.ds(col, 16)] = z16
            rows[1][r, pl.ds(col, 16)] = z16
            return carry

        lax.fori_loop(0, K * 8, zb_body, 0, unroll=8)

        def zcopy_body(j, carry):
            pltpu.sync_copy(rows[0], acc_sh.at[pl.ds(s * RPT + (2 * j) * K, K)])
            pltpu.sync_copy(rows[1], acc_sh.at[pl.ds(s * RPT + (2 * j + 1) * K, K)])
            return carry

        lax.fori_loop(0, RPT // (2 * K), zcopy_body, 0)
        plsc.subcore_barrier()

        def i_start(i, b):
            pltpu.async_copy(src_hbm.at[pl.ds(wid * EPW + i * K, K)],
                             sidx_v.at[pl.ds(b * K, K)], isem[b])

        def i_wait(i, b):
            pltpu.make_async_copy(src_hbm.at[pl.ds(wid * EPW + i * K, K)],
                                  sidx_v.at[pl.ds(b * K, K)], isem[b]).wait()

        def g_start(i, b):
            pltpu.async_copy(x_hbm.at[sidx_v.at[pl.ds(b * K, K)]],
                             rows[b], gsem[b])
            pltpu.async_copy(dst_hbm.at[pl.ds(wid * EPW + i * K, K)],
                             didx_v.at[b], dsem[b])

        def g_wait(i, b):
            pltpu.make_async_copy(x_hbm.at[sidx_v.at[pl.ds(b * K, K)]],
                                  rows[b], gsem[b]).wait()
            pltpu.make_async_copy(dst_hbm.at[pl.ds(wid * EPW + i * K, K)],
                                  didx_v.at[b], dsem[b]).wait()

        def s_start(i, b):
            pltpu.async_copy(rows[b], acc_sh.at[didx_v.at[b]], ssem[b],
                             add=True)

        def s_wait(i, b):
            pltpu.make_async_copy(rows[b], acc_sh.at[didx_v.at[b]],
                                  ssem[b]).wait()

        for t in range(B - 2):
            i_start(t, t)
            i_wait(t, t)
            g_start(t, t)
        i_start(B - 2, B - 2)

        def ring_body(j, carry):
            for t in range(B):
                i = B * j + t
                b = t
                f = (t + B - 1) % B
                g = (t + B - 2) % B
                g_wait(i, b)
                s_start(i, b)

                @pl.when(jnp.logical_and(i > 0, i < NCHUNK - B + 1))
                def _():
                    s_wait(i - 1, f)

                @pl.when(i < NCHUNK - B + 1)
                def _():
                    i_start(i + B - 1, f)

                @pl.when(i < NCHUNK - B + 2)
                def _():
                    i_wait(i + B - 2, g)
                    g_start(i + B - 2, g)
            return carry

        lax.fori_loop(0, NCHUNK // B, ring_body, 0)
        for t in range(B):
            i = NCHUNK - B + t
            s_wait(i, i % B)
        plsc.subcore_barrier()

        pltpu.sync_copy(acc_sh.at[pl.ds(s * RPT, RPT)],
                        out_hbm.at[c, pl.ds(s * RPT, RPT)])

    return agg_kernel(x, src_flat, dst_flat)


def _tc_layer_body(eps_ref, x_ref, p_ref, wa_ref, ba_ref, wb_ref, bb_ref,
                   g_ref, beta_ref, o_ref):
    h = eps_ref[0, 0] * x_ref[...] + p_ref[0, :N, :] + p_ref[1, :N, :]
    t = jnp.maximum(
        jnp.dot(h, wa_ref[...], preferred_element_type=jnp.float32)
        + ba_ref[...], 0.0)
    u = (jnp.dot(t, wb_ref[...], preferred_element_type=jnp.float32)
         + bb_ref[...])
    m = jnp.mean(u, axis=0, keepdims=True)
    v = jnp.mean((u - m) ** 2, axis=0, keepdims=True)
    o_ref[...] = jnp.maximum(
        (u - m) * lax.rsqrt(v + 1e-5) * g_ref[...] + beta_ref[...], 0.0)


def _tc_layer(x, p, eps, Wa, ba, Wb, bb, g, beta):
    eps_s = jnp.reshape(1.0 + eps, (1, 1))
    return pl.pallas_call(
        _tc_layer_body,
        out_shape=jax.ShapeDtypeStruct((N, D), jnp.float32),
    )(eps_s, x, p, Wa, ba.reshape(1, D), Wb, bb.reshape(1, D),
      g.reshape(1, D), beta.reshape(1, D))


def _tc_final_body(eps_ref, x_ref, p_ref, wa_ref, ba_ref, wb_ref, bb_ref,
                   batch_ref, o_ref):
    h = eps_ref[0, 0] * x_ref[...] + p_ref[0, :N, :] + p_ref[1, :N, :]
    t = jnp.maximum(
        jnp.dot(h, wa_ref[...], preferred_element_type=jnp.float32)
        + ba_ref[...], 0.0)
    u = (jnp.dot(t, wb_ref[...], preferred_element_type=jnp.float32)
         + bb_ref[...])
    gids = lax.broadcasted_iota(jnp.int32, (N, G), 1)
    onehot = (batch_ref[...] == gids).astype(jnp.float32)
    o_ref[...] = lax.dot_general(
        onehot, u, (((0,), (0,)), ((), ())),
        preferred_element_type=jnp.float32)


def _tc_final(x, p, eps, Wa, ba, Wb, bb, batch):
    eps_s = jnp.reshape(1.0 + eps, (1, 1))
    return pl.pallas_call(
        _tc_final_body,
        out_shape=jax.ShapeDtypeStruct((G, D), jnp.float32),
    )(eps_s, x, p, Wa, ba.reshape(1, D), Wb, bb.reshape(1, D),
      batch.reshape(N, 1))


def kernel(x, edge_index, batch,
           eps0, W0a, b0a, W0b, b0b, g0, beta0,
           eps1, W1a, b1a, W1b, b1b, g1, beta1,
           eps2, W2a, b2a, W2b, b2b):
    src = edge_index[0]
    dst = edge_index[1]

    p0 = _sc_agg(x, src, dst)
    x1 = _tc_layer(x, p0, eps0, W0a, b0a, W0b, b0b, g0, beta0)
    p1 = _sc_agg(x1, src, dst)
    x2 = _tc_layer(x1, p1, eps1, W1a, b1a, W1b, b1b, g1, beta1)
    p2 = _sc_agg(x2, src, dst)
    return _tc_final(x2, p2, eps2, W2a, b2a, W2b, b2b, batch)


# speedup vs baseline: 1.0654x; 1.0115x over previous
import functools

import jax
import jax.numpy as jnp
from jax import lax
from jax.experimental import pallas as pl
from jax.experimental.pallas import tpu as pltpu
from jax.experimental.pallas import tpu_sc as plsc

N = 10000
E = 320000
D = 128
G = 64

NC = 2
NS = 16
NW = NC * NS
EPW = E // NW
K = 40
NCHUNK = EPW // K
B = 5
NPAD = 10240
RPT = NPAD // NS


def _sc_agg(x, src_flat, dst_flat):
    mesh = plsc.VectorSubcoreMesh(core_axis_name="c", subcore_axis_name="s")

    @functools.partial(
        pl.kernel,
        mesh=mesh,
        out_type=jax.ShapeDtypeStruct((NC, NPAD, D), jnp.float32),
        scratch_types=(
            [pltpu.VMEM((B * K,), jnp.int32)]
            + [pltpu.VMEM((B, K), jnp.int32)]
            + [pltpu.VMEM((K, D), jnp.float32) for _ in range(B)]
            + [pltpu.VMEM_SHARED((NPAD, D), jnp.float32)]
            + [pltpu.SemaphoreType.DMA for _ in range(4 * B)]
        ),
    )
    def agg_kernel(x_hbm, src_hbm, dst_hbm, out_hbm, sidx_v, didx_v, *rest):
        rows = rest[0:B]
        acc_sh = rest[B]
        gsem = rest[B + 1:B + 1 + B]
        ssem = rest[B + 1 + B:B + 1 + 2 * B]
        dsem = rest[B + 1 + 2 * B:B + 1 + 3 * B]
        isem = rest[B + 1 + 3 * B:B + 1 + 4 * B]
        c = lax.axis_index("c")
        s = lax.axis_index("s")
        wid = c * NS + s

        def i_start(i, b):
            pltpu.async_copy(src_hbm.at[pl.ds(wid * EPW + i * K, K)],
                             sidx_v.at[pl.ds(b * K, K)], isem[b])

        def i_wait(i, b):
            pltpu.make_async_copy(src_hbm.at[pl.ds(wid * EPW + i * K, K)],
                                  sidx_v.at[pl.ds(b * K, K)], isem[b]).wait()

        def g_start(i, b):
            pltpu.async_copy(x_hbm.at[sidx_v.at[pl.ds(b * K, K)]],
                             rows[b], gsem[b])
            pltpu.async_copy(dst_hbm.at[pl.ds(wid * EPW + i * K, K)],
                             didx_v.at[b], dsem[b])

        def g_wait(i, b):
            pltpu.make_async_copy(x_hbm.at[sidx_v.at[pl.ds(b * K, K)]],
                                  rows[b], gsem[b]).wait()
            pltpu.make_async_copy(dst_hbm.at[pl.ds(wid * EPW + i * K, K)],
                                  didx_v.at[b], dsem[b]).wait()

        def s_start(i, b):
            pltpu.async_copy(rows[b], acc_sh.at[didx_v.at[b]], ssem[b],
                             add=True)

        def s_wait(i, b):
            pltpu.make_async_copy(rows[b], acc_sh.at[didx_v.at[b]],
                                  ssem[b]).wait()

        for t in range(B - 1):
            i_start(t, t)

        z16 = jnp.zeros((16,), jnp.float32)

        def zb_body(i, carry):
            r = i // 8
            col = (i % 8) * 16
            rows[0][r, pl.ds(col, 16)] = z16
            return carry

        lax.fori_loop(0, K * 8, zb_body, 0, unroll=8)

        def z_copy(j):
            return pltpu.make_async_copy(
                rows[0], acc_sh.at[pl.ds(s * RPT + j * K, K)], dsem[B - 1])

        for j in range(RPT // K):
            z_copy(j).start()
        for t in range(1, B - 2):
            i_wait(t, t)
            g_start(t, t)
        for j in range(RPT // K):
            z_copy(j).wait()
        i_wait(0, 0)
        g_start(0, 0)
        plsc.subcore_barrier()

        def ring_body(j, carry):
            for t in range(B):
                i = B * j + t
                b = t
                f = (t + B - 1) % B
                g = (t + B - 2) % B
                g_wait(i, b)
                s_start(i, b)

                @pl.when(jnp.logical_and(i > 0, i < NCHUNK - B + 1))
                def _():
                    s_wait(i - 1, f)

                @pl.when(i < NCHUNK - B + 1)
                def _():
                    i_start(i + B - 1, f)

                @pl.when(i < NCHUNK - B + 2)
                def _():
                    i_wait(i + B - 2, g)
                    g_start(i + B - 2, g)
            return carry

        lax.fori_loop(0, NCHUNK // B, ring_body, 0)
        for t in range(B):
            i = NCHUNK - B + t
            s_wait(i, i % B)
        plsc.subcore_barrier()

        pltpu.sync_copy(acc_sh.at[pl.ds(s * RPT, RPT)],
                        out_hbm.at[c, pl.ds(s * RPT, RPT)])

    return agg_kernel(x, src_flat, dst_flat)


def _tc_layer_body(eps_ref, x_ref, p_ref, wa_ref, ba_ref, wb_ref, bb_ref,
                   g_ref, beta_ref, o_ref):
    h = eps_ref[0, 0] * x_ref[...] + p_ref[0, :N, :] + p_ref[1, :N, :]
    t = jnp.maximum(
        jnp.dot(h, wa_ref[...], preferred_element_type=jnp.float32)
        + ba_ref[...], 0.0)
    u = (jnp.dot(t, wb_ref[...], preferred_element_type=jnp.float32)
         + bb_ref[...])
    m = jnp.mean(u, axis=0, keepdims=True)
    v = jnp.mean((u - m) ** 2, axis=0, keepdims=True)
    o_ref[...] = jnp.maximum(
        (u - m) * lax.rsqrt(v + 1e-5) * g_ref[...] + beta_ref[...], 0.0)


def _tc_layer(x, p, eps, Wa, ba, Wb, bb, g, beta):
    eps_s = jnp.reshape(1.0 + eps, (1, 1))
    return pl.pallas_call(
        _tc_layer_body,
        out_shape=jax.ShapeDtypeStruct((N, D), jnp.float32),
    )(eps_s, x, p, Wa, ba.reshape(1, D), Wb, bb.reshape(1, D),
      g.reshape(1, D), beta.reshape(1, D))


def _tc_final_body(eps_ref, x_ref, p_ref, wa_ref, ba_ref, wb_ref, bb_ref,
                   batch_ref, o_ref):
    h = eps_ref[0, 0] * x_ref[...] + p_ref[0, :N, :] + p_ref[1, :N, :]
    t = jnp.maximum(
        jnp.dot(h, wa_ref[...], preferred_element_type=jnp.float32)
        + ba_ref[...], 0.0)
    u = (jnp.dot(t, wb_ref[...], preferred_element_type=jnp.float32)
         + bb_ref[...])
    gids = lax.broadcasted_iota(jnp.int32, (N, G), 1)
    onehot = (batch_ref[...] == gids).astype(jnp.float32)
    o_ref[...] = lax.dot_general(
        onehot, u, (((0,), (0,)), ((), ())),
        preferred_element_type=jnp.float32)


def _tc_final(x, p, eps, Wa, ba, Wb, bb, batch):
    eps_s = jnp.reshape(1.0 + eps, (1, 1))
    return pl.pallas_call(
        _tc_final_body,
        out_shape=jax.ShapeDtypeStruct((G, D), jnp.float32),
    )(eps_s, x, p, Wa, ba.reshape(1, D), Wb, bb.reshape(1, D),
      batch.reshape(N, 1))


def kernel(x, edge_index, batch,
           eps0, W0a, b0a, W0b, b0b, g0, beta0,
           eps1, W1a, b1a, W1b, b1b, g1, beta1,
           eps2, W2a, b2a, W2b, b2b):
    src = edge_index[0]
    dst = edge_index[1]

    p0 = _sc_agg(x, src, dst)
    x1 = _tc_layer(x, p0, eps0, W0a, b0a, W0b, b0b, g0, beta0)
    p1 = _sc_agg(x1, src, dst)
    x2 = _tc_layer(x1, p1, eps1, W1a, b1a, W1b, b1b, g1, beta1)
    p2 = _sc_agg(x2, src, dst)
    return _tc_final(x2, p2, eps2, W2a, b2a, W2b, b2b, batch)
